# baseline (speedup 1.0000x reference)
import jax
import jax.numpy as jnp
from jax import lax
from jax.experimental import pallas as pl
from jax.experimental.pallas import tpu as pltpu
from jax.experimental.pallas import tpu_sc as plsc

_N_SRC = 10000
_N_DST = 8192
_E = 320000
_T = 4
_D_IN = 128
_D_H = 64

_NC = 2
_NS = 16
_NW = _NC * _NS
_EPW = _E // _NW
_CB = 80
_NCHUNK = _EPW // _CB
_RPS = _N_DST // _NS


def _sigmoid(x):
    return 1.0 / (1.0 + jnp.exp(-x))


def _edge_lstm_block(ef_ref, st_ref, len_ref, wbd_ref, bias_ref, rw_ref,
                     brep_ref, whht_ref, w2t_ref, eob_ref, out_ref):
    ef = ef_ref[...]
    st = st_ref[...]
    v2 = jnp.sin(jnp.dot(st, rw_ref[...], preferred_element_type=jnp.float32)
                 + brep_ref[...])
    xall = jnp.concatenate([ef, st, v2], axis=1)
    xw = jnp.dot(xall.astype(jnp.bfloat16), wbd_ref[...],
                 preferred_element_type=jnp.float32) + bias_ref[...]
    b = ef.shape[0]
    h = jnp.zeros((b, _D_H), jnp.float32)
    c = jnp.zeros((b, _D_H), jnp.float32)
    lens = len_ref[...]
    whht = whht_ref[...]
    for t in range(_T):
        g = xw[:, 256 * t:256 * (t + 1)] + jnp.dot(
            h.astype(jnp.bfloat16), whht, preferred_element_type=jnp.float32)
        i = _sigmoid(g[:, 0:64])
        f = _sigmoid(g[:, 64:128])
        gg = jnp.tanh(g[:, 128:192])
        o = _sigmoid(g[:, 192:256])
        cn = f * c + i * gg
        hn = o * jnp.tanh(cn)
        m = t < lens
        h = jnp.where(m, hn, h)
        c = jnp.where(m, cn, c)
    out_ref[...] = jnp.dot(h.astype(jnp.bfloat16), w2t_ref[...],
                           preferred_element_type=jnp.float32) + eob_ref[...]


def _proj_block(feat_ref, w1t_ref, out_ref):
    out_ref[...] = jnp.dot(feat_ref[...], w1t_ref[...],
                           preferred_element_type=jnp.float32)


def _node_block(p0_ref, p1_ref, proj_ref, feat_ref, subg_ref, eob_ref,
                nu1_ref, nu2_ref, nub_ref, fct_ref, fcb_ref, out_ref):
    agg = p0_ref[...][:, :_D_H] + p1_ref[...][:, :_D_H]
    hd = (agg - (proj_ref[...][:, :_D_H] + eob_ref[...])) * subg_ref[...]
    h2 = (jnp.dot(feat_ref[...], nu1_ref[...],
                  preferred_element_type=jnp.float32)
          + jnp.dot(hd, nu2_ref[...], preferred_element_type=jnp.float32)
          + nub_ref[...])
    h2 = jnp.maximum(h2, 0.0)
    out_ref[...] = jnp.dot(h2, fct_ref[...],
                           preferred_element_type=jnp.float32) + fcb_ref[...]


def _sc_body(proj_hbm, eproj_hbm, esrc_hbm, edst_hbm, zeros_hbm, out_hbm,
             acc_sh, sidx, didx, rows, ep, sem):
    c = lax.axis_index("c")
    s = lax.axis_index("s")
    wid = c * _NS + s
    pltpu.sync_copy(zeros_hbm, acc_sh.at[pl.ds(s * _RPS, _RPS)])
    plsc.subcore_barrier()
    base = wid * _EPW

    def chunk(j, carry):
        off = base + j * _CB
        pltpu.sync_copy(esrc_hbm.at[pl.ds(off, _CB)], sidx)
        pltpu.sync_copy(edst_hbm.at[pl.ds(off, _CB)], didx)
        pltpu.sync_copy(eproj_hbm.at[pl.ds(off, _CB)], ep)
        pltpu.async_copy(proj_hbm.at[sidx], rows, sem).wait()

        def body(i, carry2):
            for k in range(_D_H // 16):
                sl = pl.ds(k * 16, 16)
                rows[i, sl] = jnp.maximum(rows[i, sl] + ep[i, sl], 0.0)
            return carry2
        lax.fori_loop(0, _CB, body, 0)
        pltpu.sync_copy(rows, acc_sh.at[didx], add=True)
        return carry

    lax.fori_loop(0, _NCHUNK, chunk, 0)
    plsc.subcore_barrier()
    pltpu.sync_copy(acc_sh.at[pl.ds(s * _RPS, _RPS)],
                    out_hbm.at[c, pl.ds(s * _RPS, _RPS)])


def kernel(features, edge_features, seq_times, subg_norm, t_w0, t_b0, t_w, t_b,
           Wih, Whh, bih, bhh, eo_W, eo_b, nu_W, nu_b, fc_W, fc_b,
           edge_src, edge_dst, e_len):
    f32 = jnp.float32
    WihT = Wih.T
    Wbd = jnp.zeros((48, 1024), f32)
    for t in range(_T):
        cb = slice(256 * t, 256 * (t + 1))
        Wbd = Wbd.at[4 * t:4 * (t + 1), cb].set(WihT[0:4])
        Wbd = Wbd.at[16 + t, cb].set(t_w0[0, 0] * WihT[4])
        Wbd = Wbd.at[20 + 7 * t:20 + 7 * (t + 1), cb].set(WihT[5:12])
    bias_blk = bih + bhh + t_b0[0] * WihT[4]
    bias_all = jnp.tile(bias_blk, _T)[None, :]
    Rw = jnp.zeros((4, 28), f32)
    for t in range(_T):
        Rw = Rw.at[t, 7 * t:7 * (t + 1)].set(t_w[0])
    brep = jnp.tile(t_b, _T)[None, :]
    WhhT = Whh.T
    W1T = eo_W[:, :_D_IN].T
    W2T = eo_W[:, _D_IN:].T
    eob = eo_b[None, :]
    nu1T = nu_W[:, :_D_IN].T
    nu2T = nu_W[:, _D_IN:].T
    nub = nu_b[None, :]
    fcT = fc_W.T
    fcb = fc_b[None, :]

    ef2 = edge_features.reshape(_E, _T * 4)
    lens2 = e_len.astype(jnp.int32).reshape(_E, 1)
    esrc = edge_src.astype(jnp.int32)
    edst = edge_dst.astype(jnp.int32)

    be = 1280
    e_proj = pl.pallas_call(
        _edge_lstm_block,
        grid=(_E // be,),
        in_specs=[
            pl.BlockSpec((be, 16), lambda i: (i, 0)),
            pl.BlockSpec((be, 4), lambda i: (i, 0)),
            pl.BlockSpec((be, 1), lambda i: (i, 0)),
            pl.BlockSpec((48, 1024), lambda i: (0, 0)),
            pl.BlockSpec((1, 1024), lambda i: (0, 0)),
            pl.BlockSpec((4, 28), lambda i: (0, 0)),
            pl.BlockSpec((1, 28), lambda i: (0, 0)),
            pl.BlockSpec((64, 256), lambda i: (0, 0)),
            pl.BlockSpec((64, 64), lambda i: (0, 0)),
            pl.BlockSpec((1, 64), lambda i: (0, 0)),
        ],
        out_specs=pl.BlockSpec((be, 64), lambda i: (i, 0)),
        out_shape=jax.ShapeDtypeStruct((_E, _D_H), f32),
        compiler_params=pltpu.CompilerParams(
            dimension_semantics=("arbitrary",)),
    )(ef2, seq_times, lens2, Wbd.astype(jnp.bfloat16), bias_all, Rw, brep,
      WhhT.astype(jnp.bfloat16), W2T.astype(jnp.bfloat16), eob)

    W1Tp = jnp.zeros((_D_IN, 128), f32).at[:, :_D_H].set(W1T)
    bp = 2000
    proj = pl.pallas_call(
        _proj_block,
        grid=(_N_SRC // bp,),
        in_specs=[pl.BlockSpec((bp, _D_IN), lambda i: (i, 0)),
                  pl.BlockSpec((_D_IN, 128), lambda i: (0, 0))],
        out_specs=pl.BlockSpec((bp, 128), lambda i: (i, 0)),
        out_shape=jax.ShapeDtypeStruct((_N_SRC, 128), f32),
    )(features, W1Tp)

    zeros_blk = jnp.zeros((_RPS, 128), f32)
    mesh = plsc.VectorSubcoreMesh(core_axis_name="c", subcore_axis_name="s",
                                  num_cores=_NC, num_subcores=_NS)
    partials = pl.kernel(
        _sc_body,
        out_type=jax.ShapeDtypeStruct((_NC, _N_DST, 128), f32),
        mesh=mesh,
        scratch_types=[
            pltpu.VMEM_SHARED((_N_DST, 128), f32),
            pltpu.VMEM((_CB,), jnp.int32),
            pltpu.VMEM((_CB,), jnp.int32),
            pltpu.VMEM((_CB, 128), f32),
            pltpu.VMEM((_CB, _D_H), f32),
            pltpu.SemaphoreType.DMA,
        ],
    )(proj, e_proj, esrc, edst, zeros_blk)

    bn = 512
    out = pl.pallas_call(
        _node_block,
        grid=(_N_DST // bn,),
        in_specs=[
            pl.BlockSpec((bn, 128), lambda i: (i, 0)),
            pl.BlockSpec((bn, 128), lambda i: (i, 0)),
            pl.BlockSpec((bn, 128), lambda i: (i, 0)),
            pl.BlockSpec((bn, _D_IN), lambda i: (i, 0)),
            pl.BlockSpec((bn, 1), lambda i: (i, 0)),
            pl.BlockSpec((1, _D_H), lambda i: (0, 0)),
            pl.BlockSpec((_D_IN, _D_H), lambda i: (0, 0)),
            pl.BlockSpec((_D_H, _D_H), lambda i: (0, 0)),
            pl.BlockSpec((1, _D_H), lambda i: (0, 0)),
            pl.BlockSpec((_D_H, 40), lambda i: (0, 0)),
            pl.BlockSpec((1, 40), lambda i: (0, 0)),
        ],
        out_specs=pl.BlockSpec((bn, 40), lambda i: (i, 0)),
        out_shape=jax.ShapeDtypeStruct((_N_DST, 40), f32),
    )(partials[0], partials[1], proj[:_N_DST], features[:_N_DST], subg_norm,
      eob, nu1T, nu2T, nub, fcT, fcb)
    return out


# speedup vs baseline: 1.0266x; 1.0266x over previous
import jax
import jax.numpy as jnp
from jax import lax
from jax.experimental import pallas as pl
from jax.experimental.pallas import tpu as pltpu
from jax.experimental.pallas import tpu_sc as plsc

_N_SRC = 10000
_N_DST = 8192
_E = 320000
_T = 4
_D_IN = 128
_D_H = 64

_NC = 2
_NS = 16
_NW = _NC * _NS
_EPW = _E // _NW
_CB = 80
_NCHUNK = _EPW // _CB
_RPS = _N_DST // _NS


def _sigmoid(x):
    return 0.5 + 0.5 * jnp.tanh(0.5 * x)


def _edge_lstm_block(ef_ref, st_ref, mk_ref, wbd_ref, bias_ref,
                     rw_ref, brep_ref, whht_ref, w2t_ref, eob_ref, out_ref):
    ef = ef_ref[...]
    st = st_ref[...]
    mk = mk_ref[...]
    v2 = jnp.sin(jnp.dot(st, rw_ref[...], preferred_element_type=jnp.float32)
                 + brep_ref[...])
    xall = jnp.concatenate([ef, st, v2], axis=1)
    xw = jnp.dot(xall, wbd_ref[...],
                 preferred_element_type=jnp.float32) + bias_ref[...]
    b = st.shape[0]
    h = jnp.zeros((b, _D_H), jnp.float32)
    c = jnp.zeros((b, _D_H), jnp.float32)
    whht = whht_ref[...]
    for t in range(_T):
        g = xw[:, 256 * t:256 * (t + 1)] + jnp.dot(
            h, whht, preferred_element_type=jnp.float32)
        i = _sigmoid(g[:, 0:64])
        f = _sigmoid(g[:, 64:128])
        gg = jnp.tanh(g[:, 128:192])
        o = _sigmoid(g[:, 192:256])
        cn = f * c + i * gg
        hn = o * jnp.tanh(cn)
        mt = mk[:, t:t + 1]
        h = h + mt * (hn - h)
        c = c + mt * (cn - c)
    out_ref[...] = jnp.dot(h, w2t_ref[...],
                           preferred_element_type=jnp.float32) + eob_ref[...]


def _proj_block(feat_ref, w1t_ref, out_ref):
    out_ref[...] = jnp.dot(feat_ref[...], w1t_ref[...],
                           preferred_element_type=jnp.float32)


def _node_block(p0_ref, p1_ref, proj_ref, feat_ref, subg_ref, eob_ref,
                nu1_ref, nu2_ref, nub_ref, fct_ref, fcb_ref, out_ref):
    agg = p0_ref[...][:, :_D_H] + p1_ref[...][:, :_D_H]
    hd = (agg - (proj_ref[...][:, :_D_H] + eob_ref[...])) * subg_ref[...]
    h2 = (jnp.dot(feat_ref[...], nu1_ref[...],
                  preferred_element_type=jnp.float32)
          + jnp.dot(hd, nu2_ref[...], preferred_element_type=jnp.float32)
          + nub_ref[...])
    h2 = jnp.maximum(h2, 0.0)
    out_ref[...] = jnp.dot(h2, fct_ref[...],
                           preferred_element_type=jnp.float32) + fcb_ref[...]


def _sc_body(proj_hbm, eproj_hbm, esrc_hbm, edst_hbm, zeros_hbm,
             out0_hbm, out1_hbm, acc_sh, sidx, didx, rows, ep, sem):
    c = lax.axis_index("c")
    s = lax.axis_index("s")
    wid = c * _NS + s
    pltpu.sync_copy(zeros_hbm, acc_sh.at[pl.ds(s * _RPS, _RPS)])
    plsc.subcore_barrier()
    base = wid * _EPW

    def chunk(j, carry):
        off = base + j * _CB
        pltpu.sync_copy(esrc_hbm.at[pl.ds(off, _CB)], sidx)
        pltpu.sync_copy(edst_hbm.at[pl.ds(off, _CB)], didx)
        pltpu.sync_copy(eproj_hbm.at[pl.ds(off, _CB)], ep)
        pltpu.async_copy(proj_hbm.at[sidx], rows, sem).wait()

        def body(i, carry2):
            for k in range(_D_H // 16):
                sl = pl.ds(k * 16, 16)
                rows[i, sl] = jnp.maximum(rows[i, sl] + ep[i, sl], 0.0)
            return carry2
        lax.fori_loop(0, _CB, body, 0)
        pltpu.sync_copy(rows, acc_sh.at[didx], add=True)
        return carry

    lax.fori_loop(0, _NCHUNK, chunk, 0)
    plsc.subcore_barrier()

    @pl.when(c == 0)
    def _drain0():
        pltpu.sync_copy(acc_sh.at[pl.ds(s * _RPS, _RPS)],
                        out0_hbm.at[pl.ds(s * _RPS, _RPS)])

    @pl.when(c == 1)
    def _drain1():
        pltpu.sync_copy(acc_sh.at[pl.ds(s * _RPS, _RPS)],
                        out1_hbm.at[pl.ds(s * _RPS, _RPS)])


def kernel(features, edge_features, seq_times, subg_norm, t_w0, t_b0, t_w, t_b,
           Wih, Whh, bih, bhh, eo_W, eo_b, nu_W, nu_b, fc_W, fc_b,
           edge_src, edge_dst, e_len):
    f32 = jnp.float32
    WihT = Wih.T
    Wbd = jnp.zeros((48, 1024), f32)
    for t in range(_T):
        cb = slice(256 * t, 256 * (t + 1))
        Wbd = Wbd.at[4 * t:4 * (t + 1), cb].set(WihT[0:4])
        Wbd = Wbd.at[16 + t, cb].set(t_w0[0, 0] * WihT[4])
        Wbd = Wbd.at[20 + 7 * t:20 + 7 * (t + 1), cb].set(WihT[5:12])
    bias_all = jnp.tile(bih + bhh + t_b0[0] * WihT[4], _T)[None, :]
    Rw = jnp.zeros((4, 28), f32)
    for t in range(_T):
        Rw = Rw.at[t, 7 * t:7 * (t + 1)].set(t_w[0])
    brep = jnp.tile(t_b, _T)[None, :]
    WhhT = Whh.T
    W1T = eo_W[:, :_D_IN].T
    W2T = eo_W[:, _D_IN:].T
    eob = eo_b[None, :]
    nu1T = nu_W[:, :_D_IN].T
    nu2T = nu_W[:, _D_IN:].T
    nub = nu_b[None, :]
    fcT = fc_W.T
    fcb = fc_b[None, :]

    esrc = edge_src.astype(jnp.int32)
    edst = edge_dst.astype(jnp.int32)
    ef2 = edge_features.reshape(_E, _T * 4)
    maskf = (jnp.arange(_T, dtype=jnp.int32)[None, :]
             < e_len.astype(jnp.int32)[:, None]).astype(f32)

    be = 1280
    e_proj = pl.pallas_call(
        _edge_lstm_block,
        grid=(_E // be,),
        in_specs=[
            pl.BlockSpec((be, 16), lambda i: (i, 0)),
            pl.BlockSpec((be, 4), lambda i: (i, 0)),
            pl.BlockSpec((be, 4), lambda i: (i, 0)),
            pl.BlockSpec((48, 1024), lambda i: (0, 0)),
            pl.BlockSpec((1, 1024), lambda i: (0, 0)),
            pl.BlockSpec((4, 28), lambda i: (0, 0)),
            pl.BlockSpec((1, 28), lambda i: (0, 0)),
            pl.BlockSpec((64, 256), lambda i: (0, 0)),
            pl.BlockSpec((64, 64), lambda i: (0, 0)),
            pl.BlockSpec((1, 64), lambda i: (0, 0)),
        ],
        out_specs=pl.BlockSpec((be, 64), lambda i: (i, 0)),
        out_shape=jax.ShapeDtypeStruct((_E, _D_H), f32),
        compiler_params=pltpu.CompilerParams(
            dimension_semantics=("arbitrary",)),
    )(ef2, seq_times, maskf, Wbd, bias_all, Rw, brep, WhhT, W2T, eob)

    W1Tp = jnp.zeros((_D_IN, 128), f32).at[:, :_D_H].set(W1T)
    bp = 2000
    proj = pl.pallas_call(
        _proj_block,
        grid=(_N_SRC // bp,),
        in_specs=[pl.BlockSpec((bp, _D_IN), lambda i: (i, 0)),
                  pl.BlockSpec((_D_IN, 128), lambda i: (0, 0))],
        out_specs=pl.BlockSpec((bp, 128), lambda i: (i, 0)),
        out_shape=jax.ShapeDtypeStruct((_N_SRC, 128), f32),
    )(features, W1Tp)

    zeros_blk = jnp.zeros((_RPS, 128), f32)
    mesh = plsc.VectorSubcoreMesh(core_axis_name="c", subcore_axis_name="s",
                                  num_cores=_NC, num_subcores=_NS)
    p0, p1 = pl.kernel(
        _sc_body,
        out_type=(jax.ShapeDtypeStruct((_N_DST, 128), f32),
                  jax.ShapeDtypeStruct((_N_DST, 128), f32)),
        mesh=mesh,
        scratch_types=[
            pltpu.VMEM_SHARED((_N_DST, 128), f32),
            pltpu.VMEM((_CB,), jnp.int32),
            pltpu.VMEM((_CB,), jnp.int32),
            pltpu.VMEM((_CB, 128), f32),
            pltpu.VMEM((_CB, _D_H), f32),
            pltpu.SemaphoreType.DMA,
        ],
    )(proj, e_proj, esrc, edst, zeros_blk)

    bn = 512
    out = pl.pallas_call(
        _node_block,
        grid=(_N_DST // bn,),
        in_specs=[
            pl.BlockSpec((bn, 128), lambda i: (i, 0)),
            pl.BlockSpec((bn, 128), lambda i: (i, 0)),
            pl.BlockSpec((bn, 128), lambda i: (i, 0)),
            pl.BlockSpec((bn, _D_IN), lambda i: (i, 0)),
            pl.BlockSpec((bn, 1), lambda i: (i, 0)),
            pl.BlockSpec((1, _D_H), lambda i: (0, 0)),
            pl.BlockSpec((_D_IN, _D_H), lambda i: (0, 0)),
            pl.BlockSpec((_D_H, _D_H), lambda i: (0, 0)),
            pl.BlockSpec((1, _D_H), lambda i: (0, 0)),
            pl.BlockSpec((_D_H, 40), lambda i: (0, 0)),
            pl.BlockSpec((1, 40), lambda i: (0, 0)),
        ],
        out_specs=pl.BlockSpec((bn, 40), lambda i: (i, 0)),
        out_shape=jax.ShapeDtypeStruct((_N_DST, 40), f32),
    )(p0, p1, proj[:_N_DST], features[:_N_DST], subg_norm,
      eob, nu1T, nu2T, nub, fcT, fcb)
    return out


# speedup vs baseline: 1.0340x; 1.0073x over previous
import jax
import jax.numpy as jnp
from jax import lax
from jax.experimental import pallas as pl
from jax.experimental.pallas import tpu as pltpu
from jax.experimental.pallas import tpu_sc as plsc

_N_SRC = 10000
_N_DST = 8192
_E = 320000
_T = 4
_D_IN = 128
_D_H = 64

_NC = 2
_NS = 16
_NW = _NC * _NS
_EPW = _E // _NW
_CB = 80
_NCHUNK = _EPW // _CB
_RPS = _N_DST // _NS


def _sigmoid(x):
    return 0.5 + 0.5 * jnp.tanh(0.5 * x)


def _edge_lstm_block(x_ref, mk_ref, ml_ref, wf_ref, bf_ref, wbd_ref, bias_ref,
                     whht_ref, w2t_ref, eob_ref, out_ref):
    x = x_ref[...]
    mk = mk_ref[...]
    x2 = ml_ref[...] * x + jnp.sin(x * wf_ref[...] + bf_ref[...])
    xw = jnp.dot(x2, wbd_ref[...],
                 preferred_element_type=jnp.float32) + bias_ref[...]
    b = x.shape[0]
    h = jnp.zeros((b, _D_H), jnp.float32)
    c = jnp.zeros((b, _D_H), jnp.float32)
    whht = whht_ref[...]
    for t in range(_T):
        g = xw[:, 256 * t:256 * (t + 1)] + jnp.dot(
            h, whht, preferred_element_type=jnp.float32)
        i = _sigmoid(g[:, 0:64])
        f = _sigmoid(g[:, 64:128])
        gg = jnp.tanh(g[:, 128:192])
        o = _sigmoid(g[:, 192:256])
        cn = f * c + i * gg
        hn = o * jnp.tanh(cn)
        mt = mk[:, t:t + 1]
        h = h + mt * (hn - h)
        c = c + mt * (cn - c)
    out_ref[...] = jnp.dot(h, w2t_ref[...],
                           preferred_element_type=jnp.float32) + eob_ref[...]


def _proj_block(feat_ref, w1t_ref, out_ref):
    out_ref[...] = jnp.dot(feat_ref[...], w1t_ref[...],
                           preferred_element_type=jnp.float32)


def _node_block(p0_ref, p1_ref, proj_ref, feat_ref, subg_ref, eob_ref,
                nu1_ref, nu2_ref, nub_ref, fct_ref, fcb_ref, out_ref):
    agg = p0_ref[...][:, :_D_H] + p1_ref[...][:, :_D_H]
    hd = (agg - (proj_ref[...][:, :_D_H] + eob_ref[...])) * subg_ref[...]
    h2 = (jnp.dot(feat_ref[...], nu1_ref[...],
                  preferred_element_type=jnp.float32)
          + jnp.dot(hd, nu2_ref[...], preferred_element_type=jnp.float32)
          + nub_ref[...])
    h2 = jnp.maximum(h2, 0.0)
    out_ref[...] = jnp.dot(h2, fct_ref[...],
                           preferred_element_type=jnp.float32) + fcb_ref[...]


def _sc_body(proj_hbm, eproj_hbm, esrc_hbm, edst_hbm, zeros_hbm,
             out0_hbm, out1_hbm, acc_sh, sidx, didx, rows, ep, sem):
    c = lax.axis_index("c")
    s = lax.axis_index("s")
    wid = c * _NS + s
    pltpu.sync_copy(zeros_hbm, acc_sh.at[pl.ds(s * _RPS, _RPS)])
    plsc.subcore_barrier()
    base = wid * _EPW

    def chunk(j, carry):
        off = base + j * _CB
        pltpu.sync_copy(esrc_hbm.at[pl.ds(off, _CB)], sidx)
        pltpu.sync_copy(edst_hbm.at[pl.ds(off, _CB)], didx)
        pltpu.sync_copy(eproj_hbm.at[pl.ds(off, _CB)], ep)
        pltpu.async_copy(proj_hbm.at[sidx], rows, sem).wait()

        def body(i, carry2):
            for k in range(_D_H // 16):
                sl = pl.ds(k * 16, 16)
                rows[i, sl] = jnp.maximum(rows[i, sl] + ep[i, sl], 0.0)
            return carry2
        lax.fori_loop(0, _CB, body, 0)
        pltpu.sync_copy(rows, acc_sh.at[didx], add=True)
        return carry

    lax.fori_loop(0, _NCHUNK, chunk, 0)
    plsc.subcore_barrier()

    @pl.when(c == 0)
    def _drain0():
        pltpu.sync_copy(acc_sh.at[pl.ds(s * _RPS, _RPS)],
                        out0_hbm.at[pl.ds(s * _RPS, _RPS)])

    @pl.when(c == 1)
    def _drain1():
        pltpu.sync_copy(acc_sh.at[pl.ds(s * _RPS, _RPS)],
                        out1_hbm.at[pl.ds(s * _RPS, _RPS)])


def kernel(features, edge_features, seq_times, subg_norm, t_w0, t_b0, t_w, t_b,
           Wih, Whh, bih, bhh, eo_W, eo_b, nu_W, nu_b, fc_W, fc_b,
           edge_src, edge_dst, e_len):
    f32 = jnp.float32
    WihT = Wih.T
    Wbd = jnp.zeros((48, 1024), f32)
    for t in range(_T):
        cb = slice(256 * t, 256 * (t + 1))
        Wbd = Wbd.at[4 * t:4 * (t + 1), cb].set(WihT[0:4])
        Wbd = Wbd.at[16 + t, cb].set(t_w0[0, 0] * WihT[4])
        Wbd = Wbd.at[20 + 7 * t:20 + 7 * (t + 1), cb].set(WihT[5:12])
    bias_all = jnp.tile(bih + bhh + t_b0[0] * WihT[4], _T)[None, :]
    maskl = jnp.concatenate([jnp.ones((20,), f32), jnp.zeros((28,), f32)])
    wfull = jnp.concatenate([jnp.zeros((20,), f32), jnp.tile(t_w[0], _T)])
    bfull = jnp.concatenate([jnp.zeros((20,), f32), jnp.tile(t_b, _T)])
    maskl, wfull, bfull = maskl[None, :], wfull[None, :], bfull[None, :]
    WhhT = Whh.T
    W1T = eo_W[:, :_D_IN].T
    W2T = eo_W[:, _D_IN:].T
    eob = eo_b[None, :]
    nu1T = nu_W[:, :_D_IN].T
    nu2T = nu_W[:, _D_IN:].T
    nub = nu_b[None, :]
    fcT = fc_W.T
    fcb = fc_b[None, :]

    esrc = edge_src.astype(jnp.int32)
    edst = edge_dst.astype(jnp.int32)
    ef2 = edge_features.reshape(_E, _T * 4)
    xcat = jnp.concatenate([ef2, seq_times,
                            jnp.repeat(seq_times, 7, axis=1)], axis=1)
    maskf = (jnp.arange(_T, dtype=jnp.int32)[None, :]
             < e_len.astype(jnp.int32)[:, None]).astype(f32)

    be = 1280
    e_proj = pl.pallas_call(
        _edge_lstm_block,
        grid=(_E // be,),
        in_specs=[
            pl.BlockSpec((be, 48), lambda i: (i, 0)),
            pl.BlockSpec((be, 4), lambda i: (i, 0)),
            pl.BlockSpec((1, 48), lambda i: (0, 0)),
            pl.BlockSpec((1, 48), lambda i: (0, 0)),
            pl.BlockSpec((1, 48), lambda i: (0, 0)),
            pl.BlockSpec((48, 1024), lambda i: (0, 0)),
            pl.BlockSpec((1, 1024), lambda i: (0, 0)),
            pl.BlockSpec((64, 256), lambda i: (0, 0)),
            pl.BlockSpec((64, 64), lambda i: (0, 0)),
            pl.BlockSpec((1, 64), lambda i: (0, 0)),
        ],
        out_specs=pl.BlockSpec((be, 64), lambda i: (i, 0)),
        out_shape=jax.ShapeDtypeStruct((_E, _D_H), f32),
        compiler_params=pltpu.CompilerParams(
            dimension_semantics=("arbitrary",)),
    )(xcat, maskf, maskl, wfull, bfull, Wbd, bias_all, WhhT, W2T, eob)

    W1Tp = jnp.zeros((_D_IN, 128), f32).at[:, :_D_H].set(W1T)
    bp = 2000
    proj = pl.pallas_call(
        _proj_block,
        grid=(_N_SRC // bp,),
        in_specs=[pl.BlockSpec((bp, _D_IN), lambda i: (i, 0)),
                  pl.BlockSpec((_D_IN, 128), lambda i: (0, 0))],
        out_specs=pl.BlockSpec((bp, 128), lambda i: (i, 0)),
        out_shape=jax.ShapeDtypeStruct((_N_SRC, 128), f32),
    )(features, W1Tp)

    zeros_blk = jnp.zeros((_RPS, 128), f32)
    mesh = plsc.VectorSubcoreMesh(core_axis_name="c", subcore_axis_name="s",
                                  num_cores=_NC, num_subcores=_NS)
    p0, p1 = pl.kernel(
        _sc_body,
        out_type=(jax.ShapeDtypeStruct((_N_DST, 128), f32),
                  jax.ShapeDtypeStruct((_N_DST, 128), f32)),
        mesh=mesh,
        scratch_types=[
            pltpu.VMEM_SHARED((_N_DST, 128), f32),
            pltpu.VMEM((_CB,), jnp.int32),
            pltpu.VMEM((_CB,), jnp.int32),
            pltpu.VMEM((_CB, 128), f32),
            pltpu.VMEM((_CB, _D_H), f32),
            pltpu.SemaphoreType.DMA,
        ],
    )(proj, e_proj, esrc, edst, zeros_blk)

    bn = 512
    out = pl.pallas_call(
        _node_block,
        grid=(_N_DST // bn,),
        in_specs=[
            pl.BlockSpec((bn, 128), lambda i: (i, 0)),
            pl.BlockSpec((bn, 128), lambda i: (i, 0)),
            pl.BlockSpec((bn, 128), lambda i: (i, 0)),
            pl.BlockSpec((bn, _D_IN), lambda i: (i, 0)),
            pl.BlockSpec((bn, 1), lambda i: (i, 0)),
            pl.BlockSpec((1, _D_H), lambda i: (0, 0)),
            pl.BlockSpec((_D_IN, _D_H), lambda i: (0, 0)),
            pl.BlockSpec((_D_H, _D_H), lambda i: (0, 0)),
            pl.BlockSpec((1, _D_H), lambda i: (0, 0)),
            pl.BlockSpec((_D_H, 40), lambda i: (0, 0)),
            pl.BlockSpec((1, 40), lambda i: (0, 0)),
        ],
        out_specs=pl.BlockSpec((bn, 40), lambda i: (i, 0)),
        out_shape=jax.ShapeDtypeStruct((_N_DST, 40), f32),
    )(p0, p1, proj[:_N_DST], features[:_N_DST], subg_norm,
      eob, nu1T, nu2T, nub, fcT, fcb)
    return out
